# baseline (speedup 1.0000x reference)
import functools

import jax
import jax.numpy as jnp
from jax import lax
from jax.experimental import pallas as pl
from jax.experimental.pallas import tpu as pltpu
from jax.experimental.pallas import tpu_sc as plsc

B, N, D = 1024, 200, 64
NC, NS = 2, 16
NW = NC * NS
BPW = B // NW
TPW = BPW * N
CHUNK = 64
NCHUNK = TPW // CHUNK
UB = 4
NTAB_ROWS = (1001, 10001, 2001, 101, 25, 8)
REPS = (32, 4, 32, 32, 32, 32)
SCALE = float(D) ** 0.5


def _sc_gather_sum(pid, nidx, aidx, item_table, ntabs, atabs):
    mesh = plsc.VectorSubcoreMesh(core_axis_name="c", subcore_axis_name="s",
                                  num_cores=NC, num_subcores=NS)

    @functools.partial(
        pl.kernel,
        out_type=(
            jax.ShapeDtypeStruct((B * N // 2, 2 * D), jnp.float32),
            jax.ShapeDtypeStruct((B, D), jnp.float32),
        ),
        mesh=mesh,
        compiler_params=pltpu.CompilerParams(use_tc_tiling_on_sc=False),
        scratch_types=[
            [[pltpu.VMEM((CHUNK,), jnp.int32) for _ in range(2)]
             for _ in range(7)],
            [pltpu.VMEM((CHUNK, D), jnp.float32) for _ in range(2)],
            [[pltpu.VMEM((CHUNK, D), jnp.float32) for _ in range(2)]
             for _ in range(6)],
            [pltpu.VMEM((CHUNK // 2, 2 * D), jnp.float32)
             for _ in range(2)],
            pltpu.VMEM((BPW,), jnp.int32),
            [pltpu.SemaphoreType.DMA for _ in range(6)],
        ],
    )
    def k(pid_hbm, n0, n1, n2, n3, n4, n5, a0, a1, a2, a3, a4, a5,
          item_hbm, t0, t1, t2, t3, t4, t5, u0, u1, u2, u3, u4, u5,
          eu_hbm, aux_hbm,
          idx_v, acc_v, tmp_v, out_v, aidx_v, sems):
        ihbm = [pid_hbm, n0, n1, n2, n3, n4, n5]
        thbm = [item_hbm, t0, t1, t2, t3, t4, t5]
        aidx_hbm = [a0, a1, a2, a3, a4, a5]
        atab_hbm = [u0, u1, u2, u3, u4, u5]
        si = sems[0:2]
        sg = sems[2:4]
        sw = sems[4:6]

        wid = lax.axis_index("s") * NC + lax.axis_index("c")
        tok0 = wid * TPW
        bat0 = wid * BPW
        ofs = [(wid % REPS[t]) * NTAB_ROWS[t] for t in range(6)]

        def fire_idx(c, p):
            for t in range(7):
                pltpu.async_copy(ihbm[t].at[pl.ds(tok0 + c * CHUNK, CHUNK)],
                                 idx_v[t][p], si[p])

        def wait_idx(c, p):
            for t in range(7):
                pltpu.make_async_copy(
                    ihbm[t].at[pl.ds(tok0 + c * CHUNK, CHUNK)],
                    idx_v[t][p], si[p]).wait()

        def offset_idx(p):
            def obody(i, _):
                sl = pl.ds(i * 16, 16)
                for t in range(6):
                    idx_v[t + 1][p][sl] = idx_v[t + 1][p][sl] + ofs[t]
                return 0
            lax.fori_loop(0, CHUNK // 16, obody, 0)

        def fire_gathers(p):
            pltpu.async_copy(item_hbm.at[idx_v[0][p]], acc_v[p], sg[p])
            for t in range(6):
                pltpu.async_copy(thbm[t + 1].at[idx_v[t + 1][p]],
                                 tmp_v[t][p], sg[p])

        def wait_gathers(p):
            pltpu.make_async_copy(item_hbm.at[idx_v[0][p]],
                                  acc_v[p], sg[p]).wait()
            for t in range(6):
                pltpu.make_async_copy(thbm[t + 1].at[idx_v[t + 1][p]],
                                      tmp_v[t][p], sg[p]).wait()

        def fire_wb(c, p):
            pltpu.async_copy(
                out_v[p],
                eu_hbm.at[pl.ds((tok0 + c * CHUNK) // 2, CHUNK // 2)], sw[p])

        def wait_wb(c, p):
            pltpu.make_async_copy(
                out_v[p],
                eu_hbm.at[pl.ds((tok0 + c * CHUNK) // 2, CHUNK // 2)],
                sw[p]).wait()

        fire_idx(0, 0)
        fire_idx(1, 1)
        wait_idx(0, 0)
        offset_idx(0)
        fire_gathers(0)

        def body(c, p):
            q = 1 - p
            wait_gathers(p)

            @pl.when(c + 1 < NCHUNK)
            def _():
                wait_idx(c + 1, q)
                offset_idx(q)

                @pl.when(c >= 1)
                def _():
                    wait_wb(c - 1, q)
                fire_gathers(q)

            @pl.when(c + 2 < NCHUNK)
            def _():
                fire_idx(c + 2, p)

            def add_rows(i, _):
                for u in range(UB):
                    r = i * UB + u
                    ro = i * (UB // 2) + u // 2
                    for kk in range(D // 16):
                        sl = pl.ds(kk * 16, 16)
                        so = pl.ds((u % 2) * D + kk * 16, 16)
                        v = ((acc_v[p][r, sl] + tmp_v[0][p][r, sl])
                             + (tmp_v[1][p][r, sl] + tmp_v[2][p][r, sl])
                             + ((tmp_v[3][p][r, sl] + tmp_v[4][p][r, sl])
                                + tmp_v[5][p][r, sl]))
                        out_v[p][ro, so] = v
                return 0
            lax.fori_loop(0, CHUNK // UB, add_rows, 0)
            fire_wb(c, p)

        def pair(i, _):
            c = pl.multiple_of(i * 2, 2)
            body(c, 0)
            body(c + 1, 1)
            return 0
        lax.fori_loop(0, NCHUNK // 2, pair, 0)

        wait_wb(NCHUNK - 2, 0)
        wait_wb(NCHUNK - 1, 1)

        for t in range(6):
            pltpu.sync_copy(aidx_hbm[t].at[pl.ds(bat0, BPW)], aidx_v)
            pltpu.sync_copy(atab_hbm[t].at[aidx_v], tmp_v[t][0].at[pl.ds(0, BPW)])

        def aux_row(r, _):
            for kk in range(D // 16):
                sl = pl.ds(kk * 16, 16)
                v = ((tmp_v[0][0][r, sl] + tmp_v[1][0][r, sl])
                     + (tmp_v[2][0][r, sl] + tmp_v[3][0][r, sl])
                     + (tmp_v[4][0][r, sl] + tmp_v[5][0][r, sl]))
                acc_v[0][r, sl] = v
            return 0
        lax.fori_loop(0, BPW, aux_row, 0)
        pltpu.sync_copy(acc_v[0].at[pl.ds(0, BPW)], aux_hbm.at[pl.ds(bat0, BPW)])

    return k(pid, *nidx, *aidx, item_table, *ntabs, *atabs)


def _matmul_body(ce_ref, w_ref, b_ref, m_ref):
    Bb = ce_ref.shape[0]
    ce = ce_ref[...]
    mm = jnp.dot(ce.reshape(Bb * N, ce.shape[-1]), w_ref[...],
                 preferred_element_type=jnp.float32)
    m_ref[...] = mm.reshape(Bb, N, D) + b_ref[...][None]


def _tc_matmul(ce, W, b):
    Bb = 16
    return pl.pallas_call(
        _matmul_body,
        grid=(B // Bb,),
        in_specs=[
            pl.BlockSpec((Bb, N, 250), lambda i: (i, 0, 0)),
            pl.BlockSpec((250, D), lambda i: (0, 0)),
            pl.BlockSpec((1, D), lambda i: (0, 0)),
        ],
        out_specs=pl.BlockSpec((Bb, N, D), lambda i: (i, 0, 0)),
        out_shape=jax.ShapeDtypeStruct((B, N, D), jnp.float32),
    )(ce, W, b)


def _combine_body(m_ref, eu_ref, aux_ref, pid_ref, plen_ref, pos_ref,
                  seq_ref, valid_ref, lens_ref, mask_ref):
    Bb = m_ref.shape[0]
    m = m_ref[...]
    eu2 = eu_ref[...]
    eu = jnp.stack([eu2[:, :D], eu2[:, D:]], axis=1).reshape(Bb, N, D)
    aux = aux_ref[...]

    inner = jnp.concatenate(
        [aux[:, None, :], eu[:, : N - 1, :] + m[:, : N - 1, :]], axis=1)

    pid = pid_ref[...]
    valid = jnp.concatenate(
        [jnp.ones((Bb, 1), jnp.float32),
         (pid[:, : N - 1] != 0).astype(jnp.float32)], axis=1)

    pos = pos_ref[...]
    seq_ref[...] = (inner * SCALE + pos[None]) * valid[..., None]
    valid_ref[...] = valid

    lens = plen_ref[...] + 1
    lens_ref[...] = lens
    mask_ref[...] = (jax.lax.broadcasted_iota(jnp.int32, (Bb, N), 1)
                     < lens).astype(jnp.int32)


def _tc_combine(m, eu_flat, aux, pid, plens, pos_table):
    Bb = 16
    grid = (B // Bb,)
    out_shapes = (
        jax.ShapeDtypeStruct((B, N, D), jnp.float32),
        jax.ShapeDtypeStruct((B, N), jnp.float32),
        jax.ShapeDtypeStruct((B, 1), jnp.int32),
        jax.ShapeDtypeStruct((B, N), jnp.int32),
    )
    return pl.pallas_call(
        _combine_body,
        grid=grid,
        in_specs=[
            pl.BlockSpec((Bb, N, D), lambda i: (i, 0, 0)),
            pl.BlockSpec((Bb * N // 2, 2 * D), lambda i: (i, 0)),
            pl.BlockSpec((Bb, D), lambda i: (i, 0)),
            pl.BlockSpec((Bb, N), lambda i: (i, 0)),
            pl.BlockSpec((Bb, 1), lambda i: (i, 0)),
            pl.BlockSpec((N, D), lambda i: (0, 0)),
        ],
        out_specs=(
            pl.BlockSpec((Bb, N, D), lambda i: (i, 0, 0)),
            pl.BlockSpec((Bb, N), lambda i: (i, 0)),
            pl.BlockSpec((Bb, 1), lambda i: (i, 0)),
            pl.BlockSpec((Bb, N), lambda i: (i, 0)),
        ),
        out_shape=out_shapes,
    )(m, eu_flat, aux, pid, plens, pos_table)


def kernel(past_lens, past_ids, category_id, created_at, words_count, age,
           hour_of_day, day_of_week, environment, deviceGroup, os, country,
           region, referrer_type, content_embedding, item_table,
           category_id_table, created_at_table, words_count_table, age_table,
           hour_of_day_table, day_of_week_table, environment_table,
           deviceGroup_table, os_table, country_table, region_table,
           referrer_type_table, pos_table, W, b):
    pid_flat = past_ids.reshape(B * N).astype(jnp.int32)
    nidx = [x.reshape(B * N).astype(jnp.int32)
            for x in (category_id, created_at, words_count, age,
                      hour_of_day, day_of_week)]
    aidx = [x.astype(jnp.int32)
            for x in (environment, deviceGroup, os, country, region,
                      referrer_type)]
    ntabs = [category_id_table, created_at_table, words_count_table,
             age_table, hour_of_day_table, day_of_week_table]
    atabs = [environment_table, deviceGroup_table, os_table, country_table,
             region_table, referrer_type_table]

    ntabs = [jnp.tile(t, (r, 1)) for t, r in zip(ntabs, REPS)]
    eu_flat, aux = _sc_gather_sum(pid_flat, nidx, aidx, item_table,
                                  ntabs, atabs)
    m = _tc_matmul(content_embedding, W, b.reshape(1, D))

    seq, valid, lens, mask = _tc_combine(
        m, eu_flat, aux, past_ids.astype(jnp.int32),
        past_lens.astype(jnp.int32).reshape(B, 1), pos_table)

    return (lens.reshape(B).astype(past_lens.dtype), seq,
            valid[..., None], mask.astype(jnp.bool_))


# speedup vs baseline: 1.0529x; 1.0529x over previous
import functools

import jax
import jax.numpy as jnp
from jax import lax
from jax.experimental import pallas as pl
from jax.experimental.pallas import tpu as pltpu
from jax.experimental.pallas import tpu_sc as plsc

B, N, D = 1024, 200, 64
NC, NS = 2, 16
NW = NC * NS
BPW = B // NW
TPW = BPW * N
CHUNK = 64
NCHUNK = TPW // CHUNK
UB = 4
NTAB_ROWS = (1001, 10001, 2001, 101, 25, 8)
REPS = (32, 4, 32, 32, 32, 32)
SCALE = float(D) ** 0.5


def _sc_gather_sum(pid, nidx, aidx, item_table, ntabs, atabs):
    mesh = plsc.VectorSubcoreMesh(core_axis_name="c", subcore_axis_name="s",
                                  num_cores=NC, num_subcores=NS)

    @functools.partial(
        pl.kernel,
        out_type=(
            jax.ShapeDtypeStruct((B * N // 2, 2 * D), jnp.float32),
            jax.ShapeDtypeStruct((B, D), jnp.float32),
        ),
        mesh=mesh,
        compiler_params=pltpu.CompilerParams(use_tc_tiling_on_sc=False),
        scratch_types=[
            [[pltpu.VMEM((CHUNK,), jnp.int32) for _ in range(2)]
             for _ in range(7)],
            [pltpu.VMEM((CHUNK, D), jnp.float32) for _ in range(2)],
            [[pltpu.VMEM((CHUNK, D), jnp.float32) for _ in range(2)]
             for _ in range(6)],
            [pltpu.VMEM((CHUNK // 2, 2 * D), jnp.float32)
             for _ in range(2)],
            pltpu.VMEM((BPW,), jnp.int32),
            [pltpu.SemaphoreType.DMA for _ in range(6)],
        ],
    )
    def k(pid_hbm, n0, n1, n2, n3, n4, n5, a0, a1, a2, a3, a4, a5,
          item_hbm, t0, t1, t2, t3, t4, t5, u0, u1, u2, u3, u4, u5,
          eu_hbm, aux_hbm,
          idx_v, acc_v, tmp_v, out_v, aidx_v, sems):
        ihbm = [pid_hbm, n0, n1, n2, n3, n4, n5]
        thbm = [item_hbm, t0, t1, t2, t3, t4, t5]
        aidx_hbm = [a0, a1, a2, a3, a4, a5]
        atab_hbm = [u0, u1, u2, u3, u4, u5]
        si = sems[0:2]
        sg = sems[2:4]
        sw = sems[4:6]

        wid = lax.axis_index("s") * NC + lax.axis_index("c")
        tok0 = wid * TPW
        bat0 = wid * BPW
        ofs = [(wid % REPS[t]) * NTAB_ROWS[t] for t in range(6)]

        def fire_idx(c, p):
            for t in range(7):
                pltpu.async_copy(ihbm[t].at[pl.ds(tok0 + c * CHUNK, CHUNK)],
                                 idx_v[t][p], si[p])

        def wait_idx(c, p):
            for t in range(7):
                pltpu.make_async_copy(
                    ihbm[t].at[pl.ds(tok0 + c * CHUNK, CHUNK)],
                    idx_v[t][p], si[p]).wait()

        def offset_idx(p):
            def obody(i, _):
                sl = pl.ds(i * 16, 16)
                for t in range(6):
                    idx_v[t + 1][p][sl] = idx_v[t + 1][p][sl] + ofs[t]
                return 0
            lax.fori_loop(0, CHUNK // 16, obody, 0)

        def fire_gathers(p):
            pltpu.async_copy(item_hbm.at[idx_v[0][p]], acc_v[p], sg[p])
            for t in range(6):
                pltpu.async_copy(thbm[t + 1].at[idx_v[t + 1][p]],
                                 tmp_v[t][p], sg[p])

        def wait_gathers(p):
            pltpu.make_async_copy(item_hbm.at[idx_v[0][p]],
                                  acc_v[p], sg[p]).wait()
            for t in range(6):
                pltpu.make_async_copy(thbm[t + 1].at[idx_v[t + 1][p]],
                                      tmp_v[t][p], sg[p]).wait()

        def fire_wb(c, p):
            pltpu.async_copy(
                out_v[p],
                eu_hbm.at[pl.ds((tok0 + c * CHUNK) // 2, CHUNK // 2)], sw[p])

        def wait_wb(c, p):
            pltpu.make_async_copy(
                out_v[p],
                eu_hbm.at[pl.ds((tok0 + c * CHUNK) // 2, CHUNK // 2)],
                sw[p]).wait()

        fire_idx(0, 0)
        fire_idx(1, 1)
        wait_idx(0, 0)
        offset_idx(0)
        fire_gathers(0)

        def body(c, p):
            q = 1 - p
            wait_gathers(p)

            @pl.when(c + 1 < NCHUNK)
            def _():
                wait_idx(c + 1, q)
                offset_idx(q)

                @pl.when(c >= 1)
                def _():
                    wait_wb(c - 1, q)
                fire_gathers(q)

            @pl.when(c + 2 < NCHUNK)
            def _():
                fire_idx(c + 2, p)

            def add_rows(i, _):
                for u in range(UB):
                    r = i * UB + u
                    ro = i * (UB // 2) + u // 2
                    for kk in range(D // 16):
                        sl = pl.ds(kk * 16, 16)
                        so = pl.ds((u % 2) * D + kk * 16, 16)
                        v = ((acc_v[p][r, sl] + tmp_v[0][p][r, sl])
                             + (tmp_v[1][p][r, sl] + tmp_v[2][p][r, sl])
                             + ((tmp_v[3][p][r, sl] + tmp_v[4][p][r, sl])
                                + tmp_v[5][p][r, sl]))
                        out_v[p][ro, so] = v
                return 0
            lax.fori_loop(0, CHUNK // UB, add_rows, 0)
            fire_wb(c, p)

        def pair(i, _):
            c = pl.multiple_of(i * 2, 2)
            body(c, 0)
            body(c + 1, 1)
            return 0
        lax.fori_loop(0, NCHUNK // 2, pair, 0)

        wait_wb(NCHUNK - 2, 0)
        wait_wb(NCHUNK - 1, 1)

        for t in range(6):
            pltpu.sync_copy(aidx_hbm[t].at[pl.ds(bat0, BPW)], aidx_v)
            pltpu.sync_copy(atab_hbm[t].at[aidx_v], tmp_v[t][0].at[pl.ds(0, BPW)])

        def aux_row(r, _):
            for kk in range(D // 16):
                sl = pl.ds(kk * 16, 16)
                v = ((tmp_v[0][0][r, sl] + tmp_v[1][0][r, sl])
                     + (tmp_v[2][0][r, sl] + tmp_v[3][0][r, sl])
                     + (tmp_v[4][0][r, sl] + tmp_v[5][0][r, sl]))
                acc_v[0][r, sl] = v
            return 0
        lax.fori_loop(0, BPW, aux_row, 0)
        pltpu.sync_copy(acc_v[0].at[pl.ds(0, BPW)], aux_hbm.at[pl.ds(bat0, BPW)])

    return k(pid, *nidx, *aidx, item_table, *ntabs, *atabs)


def _combine_body(ce_ref, eu_ref, aux_ref, pid_ref, plen_ref, pos_ref,
                  w_ref, b_ref, seq_ref, valid_ref, lens_ref, mask_ref):
    Bb = ce_ref.shape[0]
    ce = ce_ref[...]
    mm = jnp.dot(ce.reshape(Bb * N, ce.shape[-1]), w_ref[...],
                 preferred_element_type=jnp.float32)
    m = mm.reshape(Bb, N, D)
    eu2 = eu_ref[...]
    eu = jnp.stack([eu2[:, :D], eu2[:, D:]], axis=1).reshape(Bb, N, D)
    aux = aux_ref[...]
    bvec = b_ref[...]

    inner = jnp.concatenate(
        [aux[:, None, :],
         eu[:, : N - 1, :] + m[:, : N - 1, :] + bvec[None]], axis=1)

    pid = pid_ref[...]
    valid = jnp.concatenate(
        [jnp.ones((Bb, 1), jnp.float32),
         (pid[:, : N - 1] != 0).astype(jnp.float32)], axis=1)

    pos = pos_ref[...]
    seq_ref[...] = (inner * SCALE + pos[None]) * valid[..., None]
    valid_ref[...] = valid

    lens = plen_ref[...] + 1
    lens_ref[...] = lens
    mask_ref[...] = (jax.lax.broadcasted_iota(jnp.int32, (Bb, N), 1)
                     < lens).astype(jnp.int32)


def _tc_combine(ce, eu_flat, aux, pid, plens, pos_table, W, b):
    Bb = 32
    grid = (B // Bb,)
    out_shapes = (
        jax.ShapeDtypeStruct((B, N, D), jnp.float32),
        jax.ShapeDtypeStruct((B, N), jnp.float32),
        jax.ShapeDtypeStruct((B, 1), jnp.int32),
        jax.ShapeDtypeStruct((B, N), jnp.int32),
    )
    return pl.pallas_call(
        _combine_body,
        grid=grid,
        in_specs=[
            pl.BlockSpec((Bb, N, 250), lambda i: (i, 0, 0)),
            pl.BlockSpec((Bb * N // 2, 2 * D), lambda i: (i, 0)),
            pl.BlockSpec((Bb, D), lambda i: (i, 0)),
            pl.BlockSpec((Bb, N), lambda i: (i, 0)),
            pl.BlockSpec((Bb, 1), lambda i: (i, 0)),
            pl.BlockSpec((N, D), lambda i: (0, 0)),
            pl.BlockSpec((250, D), lambda i: (0, 0)),
            pl.BlockSpec((1, D), lambda i: (0, 0)),
        ],
        out_specs=(
            pl.BlockSpec((Bb, N, D), lambda i: (i, 0, 0)),
            pl.BlockSpec((Bb, N), lambda i: (i, 0)),
            pl.BlockSpec((Bb, 1), lambda i: (i, 0)),
            pl.BlockSpec((Bb, N), lambda i: (i, 0)),
        ),
        out_shape=out_shapes,
    )(ce, eu_flat, aux, pid, plens, pos_table, W, b)


def kernel(past_lens, past_ids, category_id, created_at, words_count, age,
           hour_of_day, day_of_week, environment, deviceGroup, os, country,
           region, referrer_type, content_embedding, item_table,
           category_id_table, created_at_table, words_count_table, age_table,
           hour_of_day_table, day_of_week_table, environment_table,
           deviceGroup_table, os_table, country_table, region_table,
           referrer_type_table, pos_table, W, b):
    pid_flat = past_ids.reshape(B * N).astype(jnp.int32)
    nidx = [x.reshape(B * N).astype(jnp.int32)
            for x in (category_id, created_at, words_count, age,
                      hour_of_day, day_of_week)]
    aidx = [x.astype(jnp.int32)
            for x in (environment, deviceGroup, os, country, region,
                      referrer_type)]
    ntabs = [category_id_table, created_at_table, words_count_table,
             age_table, hour_of_day_table, day_of_week_table]
    atabs = [environment_table, deviceGroup_table, os_table, country_table,
             region_table, referrer_type_table]

    ntabs = [jnp.tile(t, (r, 1)) for t, r in zip(ntabs, REPS)]
    eu_flat, aux = _sc_gather_sum(pid_flat, nidx, aidx, item_table,
                                  ntabs, atabs)

    seq, valid, lens, mask = _tc_combine(
        content_embedding, eu_flat, aux, past_ids.astype(jnp.int32),
        past_lens.astype(jnp.int32).reshape(B, 1), pos_table, W,
        b.reshape(1, D))

    return (lens.reshape(B).astype(past_lens.dtype), seq,
            valid[..., None], mask.astype(jnp.bool_))


# speedup vs baseline: 1.0540x; 1.0010x over previous
import functools

import jax
import jax.numpy as jnp
from jax import lax
from jax.experimental import pallas as pl
from jax.experimental.pallas import tpu as pltpu
from jax.experimental.pallas import tpu_sc as plsc

B, N, D = 1024, 200, 64
NC, NS = 2, 16
NW = NC * NS
BPW = B // NW
TPW = BPW * N
CHUNK = 64
NCHUNK = TPW // CHUNK
UB = 8
NTAB_ROWS = (1001, 10001, 2001, 101, 25, 8)
REPS = (32, 4, 32, 32, 32, 32)
SCALE = float(D) ** 0.5


def _sc_gather_sum(pid, nidx, aidx, item_table, ntabs, atabs):
    mesh = plsc.VectorSubcoreMesh(core_axis_name="c", subcore_axis_name="s",
                                  num_cores=NC, num_subcores=NS)

    @functools.partial(
        pl.kernel,
        out_type=(
            jax.ShapeDtypeStruct((B * N // 2, 2 * D), jnp.float32),
            jax.ShapeDtypeStruct((B, D), jnp.float32),
        ),
        mesh=mesh,
        compiler_params=pltpu.CompilerParams(use_tc_tiling_on_sc=False),
        scratch_types=[
            [[pltpu.VMEM((CHUNK,), jnp.int32) for _ in range(2)]
             for _ in range(7)],
            [pltpu.VMEM((CHUNK, D), jnp.float32) for _ in range(2)],
            [[pltpu.VMEM((CHUNK, D), jnp.float32) for _ in range(2)]
             for _ in range(6)],
            [pltpu.VMEM((CHUNK // 2, 2 * D), jnp.float32)
             for _ in range(2)],
            pltpu.VMEM((BPW,), jnp.int32),
            [pltpu.SemaphoreType.DMA for _ in range(6)],
        ],
    )
    def k(pid_hbm, n0, n1, n2, n3, n4, n5, a0, a1, a2, a3, a4, a5,
          item_hbm, t0, t1, t2, t3, t4, t5, u0, u1, u2, u3, u4, u5,
          eu_hbm, aux_hbm,
          idx_v, acc_v, tmp_v, out_v, aidx_v, sems):
        ihbm = [pid_hbm, n0, n1, n2, n3, n4, n5]
        thbm = [item_hbm, t0, t1, t2, t3, t4, t5]
        aidx_hbm = [a0, a1, a2, a3, a4, a5]
        atab_hbm = [u0, u1, u2, u3, u4, u5]
        si = sems[0:2]
        sg = sems[2:4]
        sw = sems[4:6]

        wid = lax.axis_index("s") * NC + lax.axis_index("c")
        tok0 = wid * TPW
        bat0 = wid * BPW
        ofs = [(wid % REPS[t]) * NTAB_ROWS[t] for t in range(6)]

        def fire_idx(c, p):
            for t in range(7):
                pltpu.async_copy(ihbm[t].at[pl.ds(tok0 + c * CHUNK, CHUNK)],
                                 idx_v[t][p], si[p])

        def wait_idx(c, p):
            for t in range(7):
                pltpu.make_async_copy(
                    ihbm[t].at[pl.ds(tok0 + c * CHUNK, CHUNK)],
                    idx_v[t][p], si[p]).wait()

        def offset_idx(p):
            def obody(i, _):
                sl = pl.ds(i * 16, 16)
                for t in range(6):
                    idx_v[t + 1][p][sl] = idx_v[t + 1][p][sl] + ofs[t]
                return 0
            lax.fori_loop(0, CHUNK // 16, obody, 0)

        def fire_gathers(p):
            pltpu.async_copy(item_hbm.at[idx_v[0][p]], acc_v[p], sg[p])
            for t in range(6):
                pltpu.async_copy(thbm[t + 1].at[idx_v[t + 1][p]],
                                 tmp_v[t][p], sg[p])

        def wait_gathers(p):
            pltpu.make_async_copy(item_hbm.at[idx_v[0][p]],
                                  acc_v[p], sg[p]).wait()
            for t in range(6):
                pltpu.make_async_copy(thbm[t + 1].at[idx_v[t + 1][p]],
                                      tmp_v[t][p], sg[p]).wait()

        def fire_wb(c, p):
            pltpu.async_copy(
                out_v[p],
                eu_hbm.at[pl.ds((tok0 + c * CHUNK) // 2, CHUNK // 2)], sw[p])

        def wait_wb(c, p):
            pltpu.make_async_copy(
                out_v[p],
                eu_hbm.at[pl.ds((tok0 + c * CHUNK) // 2, CHUNK // 2)],
                sw[p]).wait()

        fire_idx(0, 0)
        fire_idx(1, 1)
        wait_idx(0, 0)
        offset_idx(0)
        fire_gathers(0)

        def body(c, p):
            q = 1 - p
            wait_gathers(p)

            @pl.when(c + 1 < NCHUNK)
            def _():
                wait_idx(c + 1, q)
                offset_idx(q)

                @pl.when(c >= 1)
                def _():
                    wait_wb(c - 1, q)
                fire_gathers(q)

            @pl.when(c + 2 < NCHUNK)
            def _():
                fire_idx(c + 2, p)

            def add_rows(i, _):
                for u in range(UB):
                    r = i * UB + u
                    ro = i * (UB // 2) + u // 2
                    for kk in range(D // 16):
                        sl = pl.ds(kk * 16, 16)
                        so = pl.ds((u % 2) * D + kk * 16, 16)
                        v = ((acc_v[p][r, sl] + tmp_v[0][p][r, sl])
                             + (tmp_v[1][p][r, sl] + tmp_v[2][p][r, sl])
                             + ((tmp_v[3][p][r, sl] + tmp_v[4][p][r, sl])
                                + tmp_v[5][p][r, sl]))
                        out_v[p][ro, so] = v
                return 0
            lax.fori_loop(0, CHUNK // UB, add_rows, 0)
            fire_wb(c, p)

        def pair(i, _):
            c = pl.multiple_of(i * 2, 2)
            body(c, 0)
            body(c + 1, 1)
            return 0
        lax.fori_loop(0, NCHUNK // 2, pair, 0)

        wait_wb(NCHUNK - 2, 0)
        wait_wb(NCHUNK - 1, 1)

        for t in range(6):
            pltpu.sync_copy(aidx_hbm[t].at[pl.ds(bat0, BPW)], aidx_v)
            pltpu.sync_copy(atab_hbm[t].at[aidx_v], tmp_v[t][0].at[pl.ds(0, BPW)])

        def aux_row(r, _):
            for kk in range(D // 16):
                sl = pl.ds(kk * 16, 16)
                v = ((tmp_v[0][0][r, sl] + tmp_v[1][0][r, sl])
                     + (tmp_v[2][0][r, sl] + tmp_v[3][0][r, sl])
                     + (tmp_v[4][0][r, sl] + tmp_v[5][0][r, sl]))
                acc_v[0][r, sl] = v
            return 0
        lax.fori_loop(0, BPW, aux_row, 0)
        pltpu.sync_copy(acc_v[0].at[pl.ds(0, BPW)], aux_hbm.at[pl.ds(bat0, BPW)])

    return k(pid, *nidx, *aidx, item_table, *ntabs, *atabs)


def _combine_body(ce_ref, eu_ref, aux_ref, pid_ref, plen_ref, pos_ref,
                  w_ref, b_ref, seq_ref, valid_ref, lens_ref, mask_ref):
    Bb = ce_ref.shape[0]
    ce = ce_ref[...]
    mm = jnp.dot(ce.reshape(Bb * N, ce.shape[-1]), w_ref[...],
                 preferred_element_type=jnp.float32)
    m = mm.reshape(Bb, N, D)
    eu2 = eu_ref[...]
    eu = jnp.stack([eu2[:, :D], eu2[:, D:]], axis=1).reshape(Bb, N, D)
    aux = aux_ref[...]
    bvec = b_ref[...]

    inner = jnp.concatenate(
        [aux[:, None, :],
         eu[:, : N - 1, :] + m[:, : N - 1, :] + bvec[None]], axis=1)

    pid = pid_ref[...]
    valid = jnp.concatenate(
        [jnp.ones((Bb, 1), jnp.float32),
         (pid[:, : N - 1] != 0).astype(jnp.float32)], axis=1)

    pos = pos_ref[...]
    seq_ref[...] = (inner * SCALE + pos[None]) * valid[..., None]
    valid_ref[...] = valid

    lens = plen_ref[...] + 1
    lens_ref[...] = lens
    mask_ref[...] = (jax.lax.broadcasted_iota(jnp.int32, (Bb, N), 1)
                     < lens).astype(jnp.int32)


def _tc_combine(ce, eu_flat, aux, pid, plens, pos_table, W, b):
    Bb = 32
    grid = (B // Bb,)
    out_shapes = (
        jax.ShapeDtypeStruct((B, N, D), jnp.float32),
        jax.ShapeDtypeStruct((B, N), jnp.float32),
        jax.ShapeDtypeStruct((B, 1), jnp.int32),
        jax.ShapeDtypeStruct((B, N), jnp.int32),
    )
    return pl.pallas_call(
        _combine_body,
        grid=grid,
        in_specs=[
            pl.BlockSpec((Bb, N, 250), lambda i: (i, 0, 0)),
            pl.BlockSpec((Bb * N // 2, 2 * D), lambda i: (i, 0)),
            pl.BlockSpec((Bb, D), lambda i: (i, 0)),
            pl.BlockSpec((Bb, N), lambda i: (i, 0)),
            pl.BlockSpec((Bb, 1), lambda i: (i, 0)),
            pl.BlockSpec((N, D), lambda i: (0, 0)),
            pl.BlockSpec((250, D), lambda i: (0, 0)),
            pl.BlockSpec((1, D), lambda i: (0, 0)),
        ],
        out_specs=(
            pl.BlockSpec((Bb, N, D), lambda i: (i, 0, 0)),
            pl.BlockSpec((Bb, N), lambda i: (i, 0)),
            pl.BlockSpec((Bb, 1), lambda i: (i, 0)),
            pl.BlockSpec((Bb, N), lambda i: (i, 0)),
        ),
        out_shape=out_shapes,
    )(ce, eu_flat, aux, pid, plens, pos_table, W, b)


def kernel(past_lens, past_ids, category_id, created_at, words_count, age,
           hour_of_day, day_of_week, environment, deviceGroup, os, country,
           region, referrer_type, content_embedding, item_table,
           category_id_table, created_at_table, words_count_table, age_table,
           hour_of_day_table, day_of_week_table, environment_table,
           deviceGroup_table, os_table, country_table, region_table,
           referrer_type_table, pos_table, W, b):
    pid_flat = past_ids.reshape(B * N).astype(jnp.int32)
    nidx = [x.reshape(B * N).astype(jnp.int32)
            for x in (category_id, created_at, words_count, age,
                      hour_of_day, day_of_week)]
    aidx = [x.astype(jnp.int32)
            for x in (environment, deviceGroup, os, country, region,
                      referrer_type)]
    ntabs = [category_id_table, created_at_table, words_count_table,
             age_table, hour_of_day_table, day_of_week_table]
    atabs = [environment_table, deviceGroup_table, os_table, country_table,
             region_table, referrer_type_table]

    ntabs = [jnp.tile(t, (r, 1)) for t, r in zip(ntabs, REPS)]
    eu_flat, aux = _sc_gather_sum(pid_flat, nidx, aidx, item_table,
                                  ntabs, atabs)

    seq, valid, lens, mask = _tc_combine(
        content_embedding, eu_flat, aux, past_ids.astype(jnp.int32),
        past_lens.astype(jnp.int32).reshape(B, 1), pos_table, W,
        b.reshape(1, D))

    return (lens.reshape(B).astype(past_lens.dtype), seq,
            valid[..., None], mask.astype(jnp.bool_))
